# baseline (speedup 1.0000x reference)
import functools
import math

import jax
import jax.numpy as jnp
from jax import lax
from jax.experimental import pallas as pl
from jax.experimental.pallas import tpu as pltpu
from jax.experimental.pallas import tpu_sc as plsc

_BATCH = 4096
_WINDOW = 200
_EMBED = 64
_B = _BATCH * _WINDOW
_NC, _NS = 2, 16
_NW = _NC * _NS
_BPW = _B // _NW
_CHB = 4
_CH = _CHB * _WINDOW
_NCHUNK = _BPW // _CH
_SCALE = math.sqrt(_EMBED)
_LANES = 16
_WSIZES = (128, 72)

_mesh = plsc.VectorSubcoreMesh(core_axis_name="c", subcore_axis_name="s")


@functools.partial(
    pl.kernel,
    out_type=jax.ShapeDtypeStruct((_BATCH, _WINDOW, _EMBED), jnp.float32),
    mesh=_mesh,
    scratch_types=[
        pltpu.VMEM((_CH,), jnp.int32),
        pltpu.VMEM((_CHB, _WINDOW, _EMBED), jnp.float32),
        pltpu.VMEM((_WINDOW, _EMBED), jnp.float32),
        pltpu.SemaphoreType.DMA,
    ],
    compiler_params=pltpu.CompilerParams(use_tc_tiling_on_sc=False),
)
def _emb_pe_kernel(x_hbm, table_hbm, pos_hbm, out_hbm, idx_v, rows_v, pos_v, sem):
    wid = lax.axis_index("s") * _NC + lax.axis_index("c")
    base = wid * _BPW
    bbase = wid * (_BPW // _WINDOW)
    pltpu.sync_copy(pos_hbm, pos_v)

    @pl.loop(0, _NCHUNK)
    def _chunk(g):
        cbase = base + g * _CH
        pltpu.sync_copy(x_hbm.at[pl.ds(cbase, _CH)], idx_v)
        copies = []
        for w in range(_CHB):
            off = w * _WINDOW
            for sz in _WSIZES:
                copies.append(
                    pltpu.async_copy(
                        table_hbm.at[idx_v.at[pl.ds(off, sz)]],
                        rows_v.at[w].at[pl.ds(off - w * _WINDOW, sz)],
                        sem,
                    )
                )
                off += sz
        for c in copies:
            c.wait()

        @pl.loop(0, _WINDOW)
        def _row(j):
            for s in range(_EMBED // _LANES):
                sl = pl.ds(s * _LANES, _LANES)
                p = pos_v[j, sl]
                for w in range(_CHB):
                    rows_v[w, j, sl] = rows_v[w, j, sl] * _SCALE + p

        pltpu.sync_copy(rows_v, out_hbm.at[pl.ds(bbase + g * _CHB, _CHB)])


def kernel(x, table, pos_encoding):
    return _emb_pe_kernel(x.reshape(_B), table, pos_encoding)


# speedup vs baseline: 1.0608x; 1.0608x over previous
import functools
import math

import jax
import jax.numpy as jnp
from jax import lax
from jax.experimental import pallas as pl
from jax.experimental.pallas import tpu as pltpu
from jax.experimental.pallas import tpu_sc as plsc

_BATCH = 4096
_WINDOW = 200
_EMBED = 64
_PITCH = 128
_B = _BATCH * _WINDOW
_NC, _NS = 2, 16
_NW = _NC * _NS
_BPW = _B // _NW
_CHB = 2
_CH = _CHB * _WINDOW
_NCHUNK = _BPW // _CH
_SCALE = math.sqrt(_EMBED)
_LANES = 16
_WSIZES = (128, 72)

_mesh = plsc.VectorSubcoreMesh(core_axis_name="c", subcore_axis_name="s")


@functools.partial(
    pl.kernel,
    out_type=jax.ShapeDtypeStruct((_BATCH, _WINDOW, _EMBED), jnp.float32),
    mesh=_mesh,
    scratch_types=[
        pltpu.VMEM((_CH,), jnp.int32),
        pltpu.VMEM((_CHB, _WINDOW, _PITCH), jnp.float32),
        pltpu.VMEM((_CHB, _WINDOW, _EMBED), jnp.float32),
        pltpu.VMEM((_WINDOW, _EMBED), jnp.float32),
        pltpu.SemaphoreType.DMA,
    ],
    compiler_params=pltpu.CompilerParams(use_tc_tiling_on_sc=True),
)
def _emb_pe_kernel(x_hbm, table_hbm, pos_hbm, out_hbm,
                   idx_v, gbuf_v, obuf_v, pos_v, sem):
    wid = lax.axis_index("s") * _NC + lax.axis_index("c")
    base = wid * _BPW
    bbase = wid * (_BPW // _WINDOW)
    pltpu.sync_copy(pos_hbm, pos_v)

    @pl.loop(0, _NCHUNK)
    def _chunk(g):
        cbase = base + g * _CH
        pltpu.sync_copy(x_hbm.at[pl.ds(cbase, _CH)], idx_v)
        copies = []
        for w in range(_CHB):
            off = w * _WINDOW
            roff = 0
            for sz in _WSIZES:
                copies.append(
                    pltpu.async_copy(
                        table_hbm.at[idx_v.at[pl.ds(off, sz)]],
                        gbuf_v.at[w].at[pl.ds(roff, sz)],
                        sem,
                    )
                )
                off += sz
                roff += sz
        for c in copies:
            c.wait()

        @pl.loop(0, _WINDOW)
        def _row(j):
            for s in range(_EMBED // _LANES):
                sl = pl.ds(s * _LANES, _LANES)
                p = pos_v[j, sl]
                for w in range(_CHB):
                    obuf_v[w, j, sl] = gbuf_v[w, j, sl] * _SCALE + p

        pltpu.sync_copy(obuf_v, out_hbm.at[pl.ds(bbase + g * _CHB, _CHB)])


def kernel(x, table, pos_encoding):
    table_padded = jnp.pad(table, ((0, 0), (0, _PITCH - _EMBED)))
    return _emb_pe_kernel(x.reshape(_B), table_padded, pos_encoding)
